# Initial kernel scaffold; baseline (speedup 1.0000x reference)
import jax
import jax.numpy as jnp
from jax.experimental import pallas as pl


def kernel(x, edge_index, edge_weight, W_f, b_f, W_b, b_b):
    raise NotImplementedError("write your pallas kernel here")


# speedup vs baseline: 6.9032x; 6.9032x over previous
import functools

import jax
import jax.numpy as jnp
from jax import lax
from jax.experimental import pallas as pl
from jax.experimental.pallas import tpu as pltpu
from jax.experimental.pallas import tpu_sc as plsc

NC = 2
NT = 16
L = 16


def _rsqrt16(x):
    i = lax.bitcast_convert_type(x, jnp.int32)
    i = jnp.int32(0x5F3759DF) - (i >> 1)
    y = lax.bitcast_convert_type(i, jnp.float32)
    for _ in range(3):
        y = y * (1.5 - 0.5 * x * y * y)
    return y


def _project_tables(x_pad, w_cat, b_cat, n_pad):
    bm = 512
    nb = n_pad // bm

    def body(x_ref, w_ref, b_ref, o_ref):
        o_ref[...] = (
            jnp.dot(
                x_ref[...],
                w_ref[0],
                preferred_element_type=jnp.float32,
                precision=lax.Precision.HIGHEST,
            )
            + b_ref[0]
        )

    return pl.pallas_call(
        body,
        grid=(NC, nb),
        in_specs=[
            pl.BlockSpec((bm, 256), lambda c, i: (i, 0)),
            pl.BlockSpec((1, 256, 256), lambda c, i: (c, 0, 0)),
            pl.BlockSpec((1, 1, 256), lambda c, i: (c, 0, 0)),
        ],
        out_specs=pl.BlockSpec((bm, 256), lambda c, i: (c * nb + i, 0)),
        out_shape=jax.ShapeDtypeStruct((NC * n_pad, 256), jnp.float32),
    )(x_pad, w_cat, b_cat)


def _make_sc_kernel(n, n_pad, e, g):
    ept = e // NT
    nch = ept // g
    npt = n_pad // NT
    mesh = plsc.VectorSubcoreMesh(core_axis_name="c", subcore_axis_name="s")

    @functools.partial(
        pl.kernel,
        out_type=jax.ShapeDtypeStruct((n, 256), jnp.float32),
        mesh=mesh,
        compiler_params=pltpu.CompilerParams(needs_layout_passes=False),
        scratch_types=dict(
            acc_sh=pltpu.VMEM_SHARED((n_pad, 128), jnp.float32),
            degi_sh=pltpu.VMEM_SHARED((n_pad,), jnp.float32),
            dego_sh=pltpu.VMEM_SHARED((n_pad,), jnp.float32),
            dis_sh=pltpu.VMEM_SHARED((2, n_pad), jnp.float32),
            rowc_v=pltpu.VMEM((g,), jnp.int32),
            ewc_v=pltpu.VMEM((g,), jnp.float32),
            disi_v=pltpu.VMEM((n_pad,), jnp.float32),
            diso_v=pltpu.VMEM((n_pad,), jnp.float32),
            cmb_v=pltpu.VMEM((npt,), jnp.float32),
            dstage_v=pltpu.VMEM((2, npt), jnp.float32),
            rowsa_v=pltpu.VMEM((g, 128), jnp.float32),
            rowsb_v=pltpu.VMEM((g, 128), jnp.float32),
            ones_v=pltpu.VMEM((g,), jnp.float32),
            dst_v=pltpu.VMEM((g,), jnp.int32),
            gidx_v=pltpu.VMEM((g,), jnp.int32),
            gidxb_v=pltpu.VMEM((g,), jnp.int32),
            af_v=pltpu.VMEM((g,), jnp.float32),
            ab_v=pltpu.VMEM((g,), jnp.float32),
            sem=pltpu.SemaphoreType.DMA,
        ),
    )
    def sc_kernel(
        tables_hbm, row_hbm, col_hbm, ew_hbm, out_hbm,
        acc_sh, degi_sh, dego_sh, dis_sh,
        rowc_v, ewc_v, disi_v, diso_v, cmb_v, dstage_v,
        rowsa_v, rowsb_v, ones_v, dst_v, gidx_v, gidxb_v, af_v, ab_v, sem,
    ):
        cid = lax.axis_index("c")
        tid = lax.axis_index("s")
        e0 = tid * ept
        n0 = tid * npt
        zero16 = jnp.zeros((L,), jnp.float32)
        one16 = jnp.ones((L,), jnp.float32)
        iota16 = lax.iota(jnp.int32, L)
        zidx16 = jnp.zeros((L,), jnp.int32)

        @pl.loop(0, npt // L)
        def _(i):
            cmb_v[pl.ds(i * L, L)] = zero16

        @pl.loop(0, g // L)
        def _(i):
            ones_v[pl.ds(i * L, L)] = one16

        pltpu.sync_copy(cmb_v, degi_sh.at[pl.ds(n0, npt)])
        pltpu.sync_copy(cmb_v, dego_sh.at[pl.ds(n0, npt)])
        plsc.subcore_barrier()

        @pl.loop(0, nch)
        def _(ci):
            eb = e0 + ci * g
            pltpu.sync_copy(col_hbm.at[pl.ds(eb, g)], dst_v)
            pltpu.sync_copy(row_hbm.at[pl.ds(eb, g)], gidx_v)
            pltpu.sync_copy(ones_v, degi_sh.at[dst_v], add=True)
            pltpu.sync_copy(ones_v, dego_sh.at[gidx_v], add=True)

        plsc.subcore_barrier()

        pltpu.sync_copy(degi_sh.at[pl.ds(n0, npt)], cmb_v)

        @pl.loop(0, npt // L)
        def _(k):
            deg = cmb_v[pl.ds(k * L, L)] + 1.0
            dstage_v[0, pl.ds(k * L, L)] = _rsqrt16(deg)

        pltpu.sync_copy(dego_sh.at[pl.ds(n0, npt)], cmb_v)

        @pl.loop(0, npt // L)
        def _(k):
            deg = cmb_v[pl.ds(k * L, L)] + 1.0
            dstage_v[1, pl.ds(k * L, L)] = _rsqrt16(deg)

        pltpu.sync_copy(dstage_v.at[0, pl.ds(0, npt)], dis_sh.at[0, pl.ds(n0, npt)])
        pltpu.sync_copy(dstage_v.at[1, pl.ds(0, npt)], dis_sh.at[1, pl.ds(n0, npt)])
        plsc.subcore_barrier()

        pltpu.sync_copy(dis_sh.at[0], disi_v)
        pltpu.sync_copy(dis_sh.at[1], diso_v)
        toff = cid * n_pad

        h = g // 2

        @pl.loop(0, npt // h)
        def _(sc):
            r0 = n0 + sc * h
            pltpu.async_copy(
                tables_hbm.at[pl.ds(2 * (toff + r0), g), :], rowsa_v, sem
            ).wait()

            @pl.loop(0, h)
            def _(j):
                di = plsc.load_gather(disi_v, [zidx16 + (r0 + j)])
                do = plsc.load_gather(diso_v, [zidx16 + (r0 + j)])
                a = di * di
                b = do * do
                for k in range(8):
                    rowsb_v[j, pl.ds(k * L, L)] = (
                        a * rowsa_v[2 * j, pl.ds(k * L, L)]
                        + b * rowsa_v[2 * j + 1, pl.ds(k * L, L)]
                    )

            pltpu.sync_copy(rowsb_v.at[pl.ds(0, h), :], acc_sh.at[pl.ds(r0, h), :])

        plsc.subcore_barrier()

        @pl.loop(0, nch)
        def _(ci):
            eb = e0 + ci * g
            pltpu.sync_copy(row_hbm.at[pl.ds(eb, g)], rowc_v)
            pltpu.sync_copy(col_hbm.at[pl.ds(eb, g)], dst_v)
            pltpu.sync_copy(ew_hbm.at[pl.ds(eb, g)], ewc_v)

            @pl.loop(0, g // L)
            def _(si):
                o = si * L
                s16 = rowc_v[pl.ds(o, L)]
                d16 = dst_v[pl.ds(o, L)]
                w16 = ewc_v[pl.ds(o, L)]
                dis = plsc.load_gather(disi_v, [s16])
                did = plsc.load_gather(disi_v, [d16])
                dos = plsc.load_gather(diso_v, [s16])
                dod = plsc.load_gather(diso_v, [d16])
                af_v[pl.ds(si * L, L)] = w16 * dis * did
                ab_v[pl.ds(si * L, L)] = w16 * dos * dod
                ga = 2 * (s16 + toff)
                gidx_v[pl.ds(si * L, L)] = ga
                gidxb_v[pl.ds(si * L, L)] = ga + 1

            cp_a = pltpu.async_copy(tables_hbm.at[gidx_v], rowsa_v, sem)
            cp_b = pltpu.async_copy(tables_hbm.at[gidxb_v], rowsb_v, sem)
            cp_a.wait()
            cp_b.wait()

            @pl.loop(0, g)
            def _(j):
                af = plsc.load_gather(af_v, [zidx16 + j])
                ab = plsc.load_gather(ab_v, [zidx16 + j])
                for k in range(8):
                    rowsa_v[j, pl.ds(k * L, L)] = (
                        af * rowsa_v[j, pl.ds(k * L, L)]
                        + ab * rowsb_v[j, pl.ds(k * L, L)]
                    )

            pltpu.sync_copy(rowsa_v, acc_sh.at[dst_v], add=True)

        plsc.subcore_barrier()

        nr = jnp.minimum(npt, n - n0)

        @pl.loop(0, nr // g)
        def _(sc):
            r0 = n0 + sc * g
            pltpu.sync_copy(acc_sh.at[pl.ds(r0, g), :], rowsa_v)

            @pl.loop(0, g)
            def _(j):
                for k in range(8):
                    rowsa_v[j, pl.ds(k * L, L)] = jnp.maximum(
                        rowsa_v[j, pl.ds(k * L, L)], 0.0
                    )

            pltpu.sync_copy(
                rowsa_v, out_hbm.at[pl.ds(r0, g), pl.ds(cid * 128, 128)]
            )

    return sc_kernel


def kernel(x, edge_index, edge_weight, W_f, b_f, W_b, b_b):
    n, d = x.shape
    e = edge_index.shape[1]
    n_pad = 10240
    g = 80

    w_cat = jnp.stack(
        [
            jnp.concatenate(
                [
                    W_f[c * 128:(c + 1) * 128, :].T,
                    W_b[c * 128:(c + 1) * 128, :].T,
                ],
                axis=1,
            )
            for c in range(NC)
        ]
    )
    b_cat = jnp.stack(
        [
            jnp.concatenate(
                [b_f[c * 128:(c + 1) * 128], b_b[c * 128:(c + 1) * 128]]
            )
            for c in range(NC)
        ]
    ).reshape(NC, 1, 256)
    x_pad = jnp.pad(x, ((0, n_pad - n), (0, 0)))
    tables = _project_tables(x_pad, w_cat, b_cat, n_pad).reshape(
        2 * NC * n_pad, 128
    )

    row = edge_index[0].astype(jnp.int32)
    col = edge_index[1].astype(jnp.int32)
    ew = edge_weight.reshape(-1).astype(jnp.float32)

    sc = _make_sc_kernel(n, n_pad, e, g)
    return sc(tables, row, col, ew)
